# baseline (speedup 1.0000x reference)
import functools

import jax
import jax.numpy as jnp
from jax import lax
from jax.experimental import pallas as pl
from jax.experimental.pallas import tpu as pltpu
from jax.experimental.pallas import tpu_sc as plsc

_D = 64
_NW = 32


@jax.jit
def _gather_nn(emb_flat, x_t):
    v = emb_flat.shape[0] // _D
    d_dim = _D
    h, b = x_t.shape
    bw = b // _NW

    mesh = plsc.VectorSubcoreMesh(core_axis_name="c", subcore_axis_name="s")

    @functools.partial(
        pl.kernel,
        mesh=mesh,
        out_type=jax.ShapeDtypeStruct((h, d_dim, b), jnp.float32),
        scratch_types=[
            pltpu.VMEM((h, bw), jnp.int32),
            pltpu.VMEM((d_dim, bw), jnp.int32),
            pltpu.VMEM((d_dim, bw), jnp.float32),
            pltpu.VMEM((d_dim, bw), jnp.float32),
            pltpu.SemaphoreType.DMA,
            pltpu.SemaphoreType.DMA,
            pltpu.SemaphoreType.DMA,
            pltpu.SemaphoreType.DMA,
        ],
        compiler_params=pltpu.CompilerParams(use_tc_tiling_on_sc=False),
    )
    def k(tbl_flat, xt_hbm, out_hbm, idx_t, offs, buf0, buf1,
          gsem0, gsem1, wsem0, wsem1):
        wid = lax.axis_index("s") * 2 + lax.axis_index("c")
        base = wid * bw
        bufs = (buf0, buf1)
        gsem = (gsem0, gsem1)
        wsem = (wsem0, wsem1)
        nvec = bw // 16

        pltpu.sync_copy(xt_hbm.at[:, pl.ds(base, bw)], idx_t)

        def wait_gathers(s):
            pltpu.make_async_copy(
                out_hbm.at[0, :, pl.ds(0, bw)], bufs[s], gsem[s]).wait()

        def wait_writeback(s):
            pltpu.make_async_copy(
                bufs[s], out_hbm.at[0, :, pl.ds(0, bw)], wsem[s]).wait()

        for hh in range(h):
            s = hh % 2

            def fire(d, carry):
                for j in range(nvec):
                    sl = pl.ds(j * 16, 16)
                    offs.at[d][sl] = idx_t.at[hh][sl] + d * v
                pltpu.async_copy(
                    tbl_flat.at[offs.at[d]], bufs[s].at[d], gsem[s])
                return carry

            if hh >= 2:
                wait_writeback(s)
            lax.fori_loop(0, d_dim, fire, 0)
            wait_gathers(s)
            pltpu.async_copy(
                bufs[s], out_hbm.at[hh, :, pl.ds(base, bw)], wsem[s])
        wait_writeback((h - 2) % 2)
        wait_writeback((h - 1) % 2)

    return k(emb_flat, x_t)


def kernel(x, embedding):
    emb_flat = jnp.swapaxes(embedding, 0, 1).reshape(-1)
    out_t = _gather_nn(emb_flat, jnp.swapaxes(x, 0, 1))
    return jnp.transpose(out_t, (2, 0, 1))


# speedup vs baseline: 9.7362x; 9.7362x over previous
import functools

import jax
import jax.numpy as jnp
from jax import lax
from jax.experimental import pallas as pl
from jax.experimental.pallas import tpu as pltpu
from jax.experimental.pallas import tpu_sc as plsc

_D = 64
_DP = 128
_NW = 32
_CB = 256
_VB = 8192


def _transpose_body(in_ref, out_ref):
    out_ref[:, 0:_D] = in_ref[...].T


def _widen_table(emb_t):
    d, v = emb_t.shape
    grid = (v + _VB - 1) // _VB
    return pl.pallas_call(
        _transpose_body,
        grid=(grid,),
        in_specs=[pl.BlockSpec((d, _VB), lambda i: (0, i))],
        out_specs=pl.BlockSpec((_VB, _DP), lambda i: (i, 0)),
        out_shape=jax.ShapeDtypeStruct((v, _DP), jnp.float32),
    )(emb_t)


@jax.jit
def _gather_nn(emb_t, x_t):
    emb_pad = _widen_table(emb_t)
    h, b = x_t.shape
    bw = b // _NW
    nsub = bw // _CB
    n_chunks = h * nsub

    mesh = plsc.VectorSubcoreMesh(core_axis_name="c", subcore_axis_name="s")

    @functools.partial(
        pl.kernel,
        mesh=mesh,
        out_type=jax.ShapeDtypeStruct((b, h, _D), jnp.float32),
        scratch_types=[
            pltpu.VMEM((h, bw), jnp.int32),
            pltpu.VMEM((_CB, _DP), jnp.float32),
            pltpu.VMEM((_CB, _DP), jnp.float32),
            pltpu.SemaphoreType.DMA,
            pltpu.SemaphoreType.DMA,
            pltpu.SemaphoreType.DMA,
            pltpu.SemaphoreType.DMA,
        ],
        compiler_params=pltpu.CompilerParams(use_tc_tiling_on_sc=False),
    )
    def k(table_hbm, xt_hbm, out_hbm, idx_t, rows0, rows1,
          gsem0, gsem1, wsem0, wsem1):
        wid = lax.axis_index("s") * 2 + lax.axis_index("c")
        base = wid * bw
        rows = (rows0, rows1)
        gsem = (gsem0, gsem1)
        wsem = (wsem0, wsem1)

        pltpu.sync_copy(xt_hbm.at[:, pl.ds(base, bw)], idx_t)

        def gather(c):
            hh, half = c // nsub, c % nsub
            return pltpu.async_copy(
                table_hbm.at[idx_t.at[hh, pl.ds(half * _CB, _CB)]],
                rows[c % 2], gsem[c % 2])

        def writeback(c):
            hh, half = c // nsub, c % nsub
            return pltpu.async_copy(
                rows[c % 2].at[:, pl.ds(0, _D)],
                out_hbm.at[pl.ds(base + half * _CB, _CB), hh],
                wsem[c % 2])

        g_pending = gather(0)
        w_pending = [None, None]
        for c in range(n_chunks):
            s = c % 2
            g_pending.wait()
            if c + 1 < n_chunks:
                if w_pending[1 - s] is not None:
                    w_pending[1 - s].wait()
                g_pending = gather(c + 1)
            w_pending[s] = writeback(c)
        w_pending[(n_chunks - 2) % 2].wait()
        w_pending[(n_chunks - 1) % 2].wait()

    return k(emb_pad, x_t)


def kernel(x, embedding):
    return _gather_nn(jnp.swapaxes(embedding, 0, 1), jnp.swapaxes(x, 0, 1))


# speedup vs baseline: 11.1516x; 1.1454x over previous
import functools

import jax
import jax.numpy as jnp
from jax import lax
from jax.experimental import pallas as pl
from jax.experimental.pallas import tpu as pltpu
from jax.experimental.pallas import tpu_sc as plsc

_D = 64
_DP = 128
_NW = 32
_CB = 256
_VB = 8192


def _transpose_body(in_ref, out_ref):
    out_ref[:, 0:_D] = in_ref[...].T


def _retile_body(in_ref, out_ref):
    out_ref[...] = in_ref[...].T


def _retile_out(flat2d):
    n, m = flat2d.shape
    bk = 1024
    return pl.pallas_call(
        _retile_body,
        grid=(n // bk,),
        in_specs=[pl.BlockSpec((bk, m), lambda i: (i, 0))],
        out_specs=pl.BlockSpec((m, bk), lambda i: (0, i)),
        out_shape=jax.ShapeDtypeStruct((m, n), jnp.float32),
    )(flat2d)


def _widen_table(emb_t):
    d, v = emb_t.shape
    grid = (v + _VB - 1) // _VB
    return pl.pallas_call(
        _transpose_body,
        grid=(grid,),
        in_specs=[pl.BlockSpec((d, _VB), lambda i: (0, i))],
        out_specs=pl.BlockSpec((_VB, _DP), lambda i: (i, 0)),
        out_shape=jax.ShapeDtypeStruct((v, _DP), jnp.float32),
    )(emb_t)


@jax.jit
def _gather_nn(emb_t, x_t):
    emb_pad = _widen_table(emb_t)
    h, b = x_t.shape
    bw = b // _NW
    nsub = bw // _CB
    n_chunks = h * nsub

    mesh = plsc.VectorSubcoreMesh(core_axis_name="c", subcore_axis_name="s")

    @functools.partial(
        pl.kernel,
        mesh=mesh,
        out_type=jax.ShapeDtypeStruct((b, h * _D), jnp.float32),
        scratch_types=[
            pltpu.VMEM((h, bw), jnp.int32),
            pltpu.VMEM((_CB, _DP), jnp.float32),
            pltpu.VMEM((_CB, _DP), jnp.float32),
            pltpu.SemaphoreType.DMA,
            pltpu.SemaphoreType.DMA,
            pltpu.SemaphoreType.DMA,
            pltpu.SemaphoreType.DMA,
        ],
        compiler_params=pltpu.CompilerParams(use_tc_tiling_on_sc=False),
    )
    def k(table_hbm, xt_hbm, out_hbm, idx_t, rows0, rows1,
          gsem0, gsem1, wsem0, wsem1):
        wid = lax.axis_index("s") * 2 + lax.axis_index("c")
        base = wid * bw
        rows = (rows0, rows1)
        gsem = (gsem0, gsem1)
        wsem = (wsem0, wsem1)

        pltpu.sync_copy(xt_hbm.at[:, pl.ds(base, bw)], idx_t)

        def gather(c):
            hh, half = c // nsub, c % nsub
            return pltpu.async_copy(
                table_hbm.at[idx_t.at[hh, pl.ds(half * _CB, _CB)]],
                rows[c % 2], gsem[c % 2])

        def writeback(c):
            hh, half = c // nsub, c % nsub
            return pltpu.async_copy(
                rows[c % 2].at[:, pl.ds(0, _D)],
                out_hbm.at[pl.ds(base + half * _CB, _CB),
                           pl.ds(hh * _D, _D)],
                wsem[c % 2])

        g_pending = gather(0)
        w_pending = [None, None]
        for c in range(n_chunks):
            s = c % 2
            g_pending.wait()
            if c + 1 < n_chunks:
                if w_pending[1 - s] is not None:
                    w_pending[1 - s].wait()
                g_pending = gather(c + 1)
            w_pending[s] = writeback(c)
        w_pending[(n_chunks - 2) % 2].wait()
        w_pending[(n_chunks - 1) % 2].wait()

    out = k(emb_pad, x_t)
    out_t = _retile_out(out)
    return jnp.transpose(out_t.reshape(h, _D, b), (2, 0, 1))


def kernel(x, embedding):
    return _gather_nn(jnp.swapaxes(embedding, 0, 1), jnp.swapaxes(x, 0, 1))
